# Initial kernel scaffold; baseline (speedup 1.0000x reference)
import jax
import jax.numpy as jnp
from jax.experimental import pallas as pl


def kernel(n_feat, edge_index, e_feat, W0, a_l0, a_r0, W1, a_l1, a_r1):
    raise NotImplementedError("write your pallas kernel here")


# speedup vs baseline: 66.7980x; 66.7980x over previous
import functools

import jax
import jax.numpy as jnp
from jax import lax
from jax.experimental import pallas as pl
from jax.experimental.pallas import tpu as pltpu
from jax.experimental.pallas import tpu_sc as plsc

N = 10000
E = 320000
D = 128
H = 8
DH = 16
ROW = 144
ERW = 16

NSC = 2
NTILE = 16
NW = NSC * NTILE
EPT = E // NW
CH = 80
NCHUNK = EPT // CH
NACC = 10240
RPT = NACC // NTILE


_MESH = plsc.VectorSubcoreMesh(core_axis_name="c", subcore_axis_name="s")


@functools.partial(
    pl.kernel,
    mesh=_MESH,
    compiler_params=pltpu.CompilerParams(use_tc_tiling_on_sc=False),
    out_type=jax.ShapeDtypeStruct((NSC, NACC, ROW), jnp.float32),
    scratch_types=[
        pltpu.VMEM((CH,), jnp.int32),
        pltpu.VMEM((CH,), jnp.int32),
        pltpu.VMEM((CH, ROW), jnp.float32),
        pltpu.VMEM((CH, ERW), jnp.float32),
        pltpu.VMEM_SHARED((NACC, ROW), jnp.float32),
        pltpu.SemaphoreType.DMA,
        pltpu.SemaphoreType.DMA,
    ],
)
def _edge_kernel(src_hbm, dst_hbm, ntab_hbm, ertab_hbm, zeros_hbm, out_hbm,
                 srcv, dstv, rowbuf, erbuf, acc, sem1, sem2):
    c = lax.axis_index("c")
    s = lax.axis_index("s")
    wid = c * NTILE + s

    pltpu.sync_copy(zeros_hbm.at[pl.ds(s * RPT, RPT)],
                    acc.at[pl.ds(s * RPT, RPT)])
    plsc.subcore_barrier()

    headmask = lax.iota(jnp.int32, 16) < H
    base = wid * EPT

    def chunk_body(k, carry):
        off = base + k * CH
        pltpu.sync_copy(src_hbm.at[pl.ds(off, CH)], srcv)
        pltpu.sync_copy(dst_hbm.at[pl.ds(off, CH)], dstv)
        ga = pltpu.async_copy(ntab_hbm.at[srcv], rowbuf, sem1)
        gb = pltpu.async_copy(ertab_hbm.at[dstv], erbuf, sem2)
        ga.wait()
        gb.wait()

        def edge_body(i, ecarry):
            el16 = rowbuf[i, pl.ds(D, 16)]
            er16 = erbuf[i, :]
            ssum = el16 + er16
            e = jnp.maximum(ssum, 0.2 * ssum)
            ex = jnp.where(headmask, jnp.exp(e), 0.0)
            rowbuf[i, pl.ds(D, 16)] = ex
            for hd in range(H):
                spl = lax.gather(
                    ex, jnp.full((16, 1), hd, jnp.int32),
                    lax.GatherDimensionNumbers(offset_dims=(),
                                               collapsed_slice_dims=(0,),
                                               start_index_map=(0,)),
                    slice_sizes=(1,),
                    mode=lax.GatherScatterMode.PROMISE_IN_BOUNDS)
                rowbuf[i, pl.ds(hd * DH, DH)] = rowbuf[i, pl.ds(hd * DH, DH)] * spl
            return ecarry

        lax.fori_loop(0, CH, edge_body, 0)
        pltpu.sync_copy(rowbuf, acc.at[dstv], add=True)
        return carry

    lax.fori_loop(0, NCHUNK, chunk_body, 0)
    plsc.subcore_barrier()
    pltpu.sync_copy(acc.at[pl.ds(s * RPT, RPT)],
                    out_hbm.at[c, pl.ds(s * RPT, RPT)])


def _tables(h, gal, gar, p1, p2, p3):
    el = jnp.dot(h, gal, preferred_element_type=jnp.float32)
    er = jnp.dot(h, gar, preferred_element_type=jnp.float32)
    ntab = (jnp.dot(h, p1, preferred_element_type=jnp.float32)
            + jnp.dot(el, p2, preferred_element_type=jnp.float32))
    ertab = jnp.dot(er, p3, preferred_element_type=jnp.float32)
    return ntab, ertab


def _tables0_body(x_ref, w_ref, gal_ref, gar_ref, p1_ref, p2_ref, p3_ref,
                  ntab_ref, ertab_ref):
    h = jnp.dot(x_ref[...], w_ref[...], preferred_element_type=jnp.float32)
    ntab, ertab = _tables(h, gal_ref[...], gar_ref[...],
                          p1_ref[...], p2_ref[...], p3_ref[...])
    ntab_ref[...] = ntab
    ertab_ref[...] = ertab


def _combine(p_ref, s_ref, t_ref, gt_ref):
    p = p_ref[0][0:N] + p_ref[1][0:N]
    hp = jnp.dot(p, s_ref[...], preferred_element_type=jnp.float32)
    den = jnp.dot(p, t_ref[...], preferred_element_type=jnp.float32)
    inv = 1.0 / (den + 1e-9)
    return hp * jnp.dot(inv, gt_ref[...], preferred_element_type=jnp.float32)


def _mid_body(p_ref, s_ref, t_ref, gt_ref, w_ref, gal_ref, gar_ref,
              p1_ref, p2_ref, p3_ref, ntab_ref, ertab_ref):
    x = _combine(p_ref, s_ref, t_ref, gt_ref)
    x = jnp.maximum(x, 0.01 * x)
    h = jnp.dot(x, w_ref[...], preferred_element_type=jnp.float32)
    ntab, ertab = _tables(h, gal_ref[...], gar_ref[...],
                          p1_ref[...], p2_ref[...], p3_ref[...])
    ntab_ref[...] = ntab
    ertab_ref[...] = ertab


def _final_body(p_ref, s_ref, t_ref, gt_ref, out_ref):
    out_ref[...] = _combine(p_ref, s_ref, t_ref, gt_ref)


_tables0_call = pl.pallas_call(
    _tables0_body,
    out_shape=[jax.ShapeDtypeStruct((N, ROW), jnp.float32),
               jax.ShapeDtypeStruct((N, ERW), jnp.float32)],
)

_mid_call = pl.pallas_call(
    _mid_body,
    out_shape=[jax.ShapeDtypeStruct((N, ROW), jnp.float32),
               jax.ShapeDtypeStruct((N, ERW), jnp.float32)],
)

_final_call = pl.pallas_call(
    _final_body,
    out_shape=jax.ShapeDtypeStruct((N, D), jnp.float32),
)


def _attn_mat(a):
    r = jnp.arange(D)
    return jnp.zeros((D, H), jnp.float32).at[r, r // DH].set(a.reshape(-1))


def kernel(n_feat, edge_index, e_feat, W0, a_l0, a_r0, W1, a_l1, a_r1):
    del e_feat
    src = edge_index[0].astype(jnp.int32)
    dst = edge_index[1].astype(jnp.int32)

    r = jnp.arange(D)
    r8 = jnp.arange(H)
    one = jnp.float32(1.0)
    p1 = jnp.zeros((D, ROW), jnp.float32).at[r, r].set(one)
    p2 = jnp.zeros((H, ROW), jnp.float32).at[r8, D + r8].set(one)
    p3 = jnp.zeros((H, ERW), jnp.float32).at[r8, r8].set(one)
    smat = jnp.zeros((ROW, D), jnp.float32).at[r, r].set(one)
    tmat = jnp.zeros((ROW, H), jnp.float32).at[D + r8, r8].set(one)
    gt = jnp.zeros((H, D), jnp.float32).at[r // DH, r].set(one)
    zeros_tab = jnp.zeros((NACC, ROW), jnp.float32)

    ntab, ertab = _tables0_call(n_feat, W0, _attn_mat(a_l0), _attn_mat(a_r0),
                                p1, p2, p3)
    part1 = _edge_kernel(src, dst, ntab, ertab, zeros_tab)
    ntab2, ertab2 = _mid_call(part1, smat, tmat, gt, W1,
                              _attn_mat(a_l1), _attn_mat(a_r1), p1, p2, p3)
    part2 = _edge_kernel(src, dst, ntab2, ertab2, zeros_tab)
    return _final_call(part2, smat, tmat, gt)


# speedup vs baseline: 129.8396x; 1.9438x over previous
import functools

import jax
import jax.numpy as jnp
from jax import lax
from jax.experimental import pallas as pl
from jax.experimental.pallas import tpu as pltpu
from jax.experimental.pallas import tpu_sc as plsc

N = 10000
E = 320000
D = 128
H = 8
DH = 16
ROW = 144
ERW = 16

NSC = 2
NTILE = 16
NW = NSC * NTILE
EPT = E // NW
CH = 40
NCHUNK = EPT // CH
NBUF = 4
NIDX = 8
NACC = 10000
RPT = NACC // NTILE


_MESH = plsc.VectorSubcoreMesh(core_axis_name="c", subcore_axis_name="s")


@functools.partial(
    pl.kernel,
    mesh=_MESH,
    compiler_params=pltpu.CompilerParams(use_tc_tiling_on_sc=False),
    out_type=jax.ShapeDtypeStruct((NSC, NACC, ROW), jnp.float32),
    scratch_types=[
        [pltpu.VMEM((CH,), jnp.int32) for _ in range(NIDX)],
        [pltpu.VMEM((CH,), jnp.int32) for _ in range(NIDX)],
        [pltpu.VMEM((CH, ROW), jnp.float32) for _ in range(NBUF)],
        [pltpu.VMEM((CH, ERW), jnp.float32) for _ in range(NBUF)],
        pltpu.VMEM_SHARED((NACC, ROW), jnp.float32),
        [pltpu.SemaphoreType.DMA for _ in range(NIDX)],
        [pltpu.SemaphoreType.DMA for _ in range(NBUF)],
        [pltpu.SemaphoreType.DMA for _ in range(NBUF)],
        [pltpu.SemaphoreType.DMA for _ in range(NBUF)],
    ],
)
def _edge_kernel(src_hbm, dst_hbm, ntab_hbm, ertab_hbm, zeros_hbm, out_hbm,
                 srcbufs, dstbufs, rowbufs, erbufs, acc,
                 isems, rsems, esems, ssems):
    c = lax.axis_index("c")
    s = lax.axis_index("s")
    wid = c * NTILE + s

    pltpu.sync_copy(zeros_hbm.at[pl.ds(s * RPT, RPT)],
                    acc.at[pl.ds(s * RPT, RPT)])
    plsc.subcore_barrier()

    headmask = lax.iota(jnp.int32, 16) < H

    def issue_idx(k, ib):
        pltpu.async_copy(src_hbm.at[wid, k], srcbufs[ib], isems[ib])
        pltpu.async_copy(dst_hbm.at[wid, k], dstbufs[ib], isems[ib])

    def wait_idx(ib):
        pltpu.make_async_copy(src_hbm.at[wid, 0], srcbufs[ib],
                              isems[ib]).wait()
        pltpu.make_async_copy(dst_hbm.at[wid, 0], dstbufs[ib],
                              isems[ib]).wait()

    def issue_gathers(db, ib):
        pltpu.async_copy(ntab_hbm.at[srcbufs[ib]], rowbufs[db], rsems[db])
        pltpu.async_copy(ertab_hbm.at[dstbufs[ib]], erbufs[db], esems[db])

    def wait_gather(db):
        pltpu.make_async_copy(ntab_hbm.at[srcbufs[0]], rowbufs[db],
                              rsems[db]).wait()
        pltpu.make_async_copy(ertab_hbm.at[dstbufs[0]], erbufs[db],
                              esems[db]).wait()

    def scatter(db, ib):
        pltpu.async_copy(rowbufs[db], acc.at[dstbufs[ib]], ssems[db],
                         add=True)

    def wait_scatter(db):
        pltpu.make_async_copy(rowbufs[db], acc.at[dstbufs[0]],
                              ssems[db]).wait()

    def compute(db):
        rowbuf = rowbufs[db]
        erbuf = erbufs[db]

        def edge_body(i, ecarry):
            el16 = rowbuf[i, pl.ds(D, 16)]
            er16 = erbuf[i, :]
            ssum = el16 + er16
            e = jnp.maximum(ssum, 0.2 * ssum)
            ex = jnp.where(headmask, jnp.exp(e), 0.0)
            rowbuf[i, pl.ds(D, 16)] = ex
            for hd in range(H):
                spl = lax.gather(
                    ex, jnp.full((16, 1), hd, jnp.int32),
                    lax.GatherDimensionNumbers(offset_dims=(),
                                               collapsed_slice_dims=(0,),
                                               start_index_map=(0,)),
                    slice_sizes=(1,),
                    mode=lax.GatherScatterMode.PROMISE_IN_BOUNDS)
                rowbuf[i, pl.ds(hd * DH, DH)] = rowbuf[i, pl.ds(hd * DH, DH)] * spl
            return ecarry

        lax.fori_loop(0, CH, edge_body, 0)

    for k in range(6):
        issue_idx(k, k)
    for k in (0, 1):
        wait_idx(k)
        issue_gathers(k, k)
    for ck in (0, 1):
        issue_idx(ck + 6, ck + 6)
        wait_idx(ck + 2)
        issue_gathers(ck + 2, ck + 2)
        wait_gather(ck)
        compute(ck)
        scatter(ck, ck)

    n_groups = (NCHUNK - 2 - 8) // 8

    def group_body(g, carry):
        for j in range(8):
            ck = 2 + g * 8 + j
            db = (2 + j) % NBUF
            ib = (2 + j) % NIDX
            wait_scatter((2 + j + 2) % NBUF)
            issue_idx(ck + 6, (2 + j + 6) % NIDX)
            wait_idx((2 + j + 2) % NIDX)
            issue_gathers((2 + j + 2) % NBUF, (2 + j + 2) % NIDX)
            wait_gather(db)
            compute(db)
            scatter(db, ib)
        return carry

    lax.fori_loop(0, n_groups, group_body, 0)

    L = 2 + 8 * n_groups
    for ck in range(L, NCHUNK):
        db = ck % NBUF
        ib = ck % NIDX
        if ck + 2 < NCHUNK:
            wait_scatter((ck + 2) % NBUF)
            if ck + 6 < NCHUNK:
                issue_idx(ck + 6, (ck + 6) % NIDX)
            wait_idx((ck + 2) % NIDX)
            issue_gathers((ck + 2) % NBUF, (ck + 2) % NIDX)
        wait_gather(db)
        compute(db)
        scatter(db, ib)
    for ck in range(NCHUNK - 4, NCHUNK):
        wait_scatter(ck % NBUF)

    plsc.subcore_barrier()
    pltpu.sync_copy(acc.at[pl.ds(s * RPT, RPT)],
                    out_hbm.at[c, pl.ds(s * RPT, RPT)])


def _tables(h, gal, gar, p1, p2, p3):
    el = jnp.dot(h, gal, preferred_element_type=jnp.float32)
    er = jnp.dot(h, gar, preferred_element_type=jnp.float32)
    ntab = (jnp.dot(h, p1, preferred_element_type=jnp.float32)
            + jnp.dot(el, p2, preferred_element_type=jnp.float32))
    ertab = jnp.dot(er, p3, preferred_element_type=jnp.float32)
    return ntab, ertab


def _tables0_body(x_ref, w_ref, gal_ref, gar_ref, p1_ref, p2_ref, p3_ref,
                  ntab_ref, ertab_ref):
    h = jnp.dot(x_ref[...], w_ref[...], preferred_element_type=jnp.float32)
    ntab, ertab = _tables(h, gal_ref[...], gar_ref[...],
                          p1_ref[...], p2_ref[...], p3_ref[...])
    ntab_ref[...] = ntab
    ertab_ref[...] = ertab


def _combine(p_ref, s_ref, t_ref, gt_ref):
    p = p_ref[0][0:N] + p_ref[1][0:N]
    hp = jnp.dot(p, s_ref[...], preferred_element_type=jnp.float32)
    den = jnp.dot(p, t_ref[...], preferred_element_type=jnp.float32)
    inv = 1.0 / (den + 1e-9)
    return hp * jnp.dot(inv, gt_ref[...], preferred_element_type=jnp.float32)


def _mid_body(p_ref, s_ref, t_ref, gt_ref, w_ref, gal_ref, gar_ref,
              p1_ref, p2_ref, p3_ref, ntab_ref, ertab_ref):
    x = _combine(p_ref, s_ref, t_ref, gt_ref)
    x = jnp.maximum(x, 0.01 * x)
    h = jnp.dot(x, w_ref[...], preferred_element_type=jnp.float32)
    ntab, ertab = _tables(h, gal_ref[...], gar_ref[...],
                          p1_ref[...], p2_ref[...], p3_ref[...])
    ntab_ref[...] = ntab
    ertab_ref[...] = ertab


def _final_body(p_ref, s_ref, t_ref, gt_ref, out_ref):
    out_ref[...] = _combine(p_ref, s_ref, t_ref, gt_ref)


_tables0_call = pl.pallas_call(
    _tables0_body,
    out_shape=[jax.ShapeDtypeStruct((N, ROW), jnp.float32),
               jax.ShapeDtypeStruct((N, ERW), jnp.float32)],
)

_mid_call = pl.pallas_call(
    _mid_body,
    out_shape=[jax.ShapeDtypeStruct((N, ROW), jnp.float32),
               jax.ShapeDtypeStruct((N, ERW), jnp.float32)],
)

_final_call = pl.pallas_call(
    _final_body,
    out_shape=jax.ShapeDtypeStruct((N, D), jnp.float32),
)


def _attn_mat(a):
    r = jnp.arange(D)
    return jnp.zeros((D, H), jnp.float32).at[r, r // DH].set(a.reshape(-1))


def kernel(n_feat, edge_index, e_feat, W0, a_l0, a_r0, W1, a_l1, a_r1):
    del e_feat
    src = edge_index[0].astype(jnp.int32).reshape(NW, NCHUNK, CH)
    dst = edge_index[1].astype(jnp.int32).reshape(NW, NCHUNK, CH)

    r = jnp.arange(D)
    r8 = jnp.arange(H)
    one = jnp.float32(1.0)
    p1 = jnp.zeros((D, ROW), jnp.float32).at[r, r].set(one)
    p2 = jnp.zeros((H, ROW), jnp.float32).at[r8, D + r8].set(one)
    p3 = jnp.zeros((H, ERW), jnp.float32).at[r8, r8].set(one)
    smat = jnp.zeros((ROW, D), jnp.float32).at[r, r].set(one)
    tmat = jnp.zeros((ROW, H), jnp.float32).at[D + r8, r8].set(one)
    gt = jnp.zeros((H, D), jnp.float32).at[r // DH, r].set(one)
    zeros_tab = jnp.zeros((NACC, ROW), jnp.float32)

    ntab, ertab = _tables0_call(n_feat, W0, _attn_mat(a_l0), _attn_mat(a_r0),
                                p1, p2, p3)
    part1 = _edge_kernel(src, dst, ntab, ertab, zeros_tab)
    ntab2, ertab2 = _mid_call(part1, smat, tmat, gt, W1,
                              _attn_mat(a_l1), _attn_mat(a_r1), p1, p2, p3)
    part2 = _edge_kernel(src, dst, ntab2, ertab2, zeros_tab)
    return _final_call(part2, smat, tmat, gt)


# speedup vs baseline: 178.3946x; 1.3740x over previous
import functools

import jax
import jax.numpy as jnp
from jax import lax
from jax.experimental import pallas as pl
from jax.experimental.pallas import tpu as pltpu
from jax.experimental.pallas import tpu_sc as plsc

N = 10000
E = 320000
D = 128
H = 8
DH = 16
ROW = 144
ERW = 16

NSC = 2
NTILE = 16
NW = NSC * NTILE
EPT = E // NW
CH = 40
NCHUNK = EPT // CH
NBUF = 4
NIDX = 8
NACC = 10000
RPT = NACC // NTILE


_MESH = plsc.VectorSubcoreMesh(core_axis_name="c", subcore_axis_name="s")


@functools.partial(
    pl.kernel,
    mesh=_MESH,
    compiler_params=pltpu.CompilerParams(use_tc_tiling_on_sc=False),
    out_type=jax.ShapeDtypeStruct((NSC, NACC, ROW), jnp.float32),
    scratch_types=[
        [pltpu.VMEM((CH,), jnp.int32) for _ in range(NIDX)],
        [pltpu.VMEM((CH,), jnp.int32) for _ in range(NIDX)],
        [pltpu.VMEM((CH, ROW), jnp.float32) for _ in range(NBUF)],
        [pltpu.VMEM((CH, ERW), jnp.float32) for _ in range(NBUF)],
        pltpu.VMEM_SHARED((NACC, ROW), jnp.float32),
        [pltpu.SemaphoreType.DMA for _ in range(NIDX)],
        [pltpu.SemaphoreType.DMA for _ in range(NBUF)],
        [pltpu.SemaphoreType.DMA for _ in range(NBUF)],
        [pltpu.SemaphoreType.DMA for _ in range(NBUF)],
    ],
)
def _edge_kernel(src_hbm, dst_hbm, ntab_hbm, ertab_hbm, zeros_hbm, out_hbm,
                 srcbufs, dstbufs, rowbufs, erbufs, acc,
                 isems, rsems, esems, ssems):
    c = lax.axis_index("c")
    s = lax.axis_index("s")
    wid = c * NTILE + s

    pltpu.sync_copy(zeros_hbm.at[pl.ds(s * RPT, RPT)],
                    acc.at[pl.ds(s * RPT, RPT)])
    plsc.subcore_barrier()

    headmask = lax.iota(jnp.int32, 16) < H

    def issue_idx(k, ib):
        pltpu.async_copy(src_hbm.at[wid, k], srcbufs[ib], isems[ib])
        pltpu.async_copy(dst_hbm.at[wid, k], dstbufs[ib], isems[ib])

    def wait_idx(ib):
        pltpu.make_async_copy(src_hbm.at[wid, 0], srcbufs[ib],
                              isems[ib]).wait()
        pltpu.make_async_copy(dst_hbm.at[wid, 0], dstbufs[ib],
                              isems[ib]).wait()

    def issue_gathers(db, ib):
        pltpu.async_copy(ntab_hbm.at[srcbufs[ib]], rowbufs[db], rsems[db])
        pltpu.async_copy(ertab_hbm.at[dstbufs[ib]], erbufs[db], esems[db])

    def wait_gather(db):
        pltpu.make_async_copy(ntab_hbm.at[srcbufs[0]], rowbufs[db],
                              rsems[db]).wait()
        pltpu.make_async_copy(ertab_hbm.at[dstbufs[0]], erbufs[db],
                              esems[db]).wait()

    def scatter(db, ib):
        pltpu.async_copy(rowbufs[db], acc.at[dstbufs[ib]], ssems[db],
                         add=True)

    def wait_scatter(db):
        pltpu.make_async_copy(rowbufs[db], acc.at[dstbufs[0]],
                              ssems[db]).wait()

    def compute(db):
        rowbuf = rowbufs[db]
        erbuf = erbufs[db]

        @plsc.parallel_loop(0, CH, unroll=4)
        def edge_body(i):
            el16 = rowbuf[i, pl.ds(D, 16)]
            er16 = erbuf[i, :]
            ssum = el16 + er16
            e = jnp.maximum(ssum, 0.2 * ssum)
            ex = jnp.where(headmask, jnp.exp(e), 0.0)
            rowbuf[i, pl.ds(D, 16)] = ex
            for hd in range(H):
                spl = lax.gather(
                    ex, jnp.full((16, 1), hd, jnp.int32),
                    lax.GatherDimensionNumbers(offset_dims=(),
                                               collapsed_slice_dims=(0,),
                                               start_index_map=(0,)),
                    slice_sizes=(1,),
                    mode=lax.GatherScatterMode.PROMISE_IN_BOUNDS)
                rowbuf[i, pl.ds(hd * DH, DH)] = rowbuf[i, pl.ds(hd * DH, DH)] * spl

    for k in range(6):
        issue_idx(k, k)
    for k in (0, 1):
        wait_idx(k)
        issue_gathers(k, k)
    for ck in (0, 1):
        issue_idx(ck + 6, ck + 6)
        wait_idx(ck + 2)
        issue_gathers(ck + 2, ck + 2)
        wait_gather(ck)
        compute(ck)
        scatter(ck, ck)

    n_groups = (NCHUNK - 2 - 8) // 8

    def group_body(g, carry):
        for j in range(8):
            ck = 2 + g * 8 + j
            db = (2 + j) % NBUF
            ib = (2 + j) % NIDX
            wait_scatter((2 + j + 2) % NBUF)
            issue_idx(ck + 6, (2 + j + 6) % NIDX)
            wait_idx((2 + j + 2) % NIDX)
            issue_gathers((2 + j + 2) % NBUF, (2 + j + 2) % NIDX)
            wait_gather(db)
            compute(db)
            scatter(db, ib)
        return carry

    lax.fori_loop(0, n_groups, group_body, 0)

    L = 2 + 8 * n_groups
    for ck in range(L, NCHUNK):
        db = ck % NBUF
        ib = ck % NIDX
        if ck + 2 < NCHUNK:
            wait_scatter((ck + 2) % NBUF)
            if ck + 6 < NCHUNK:
                issue_idx(ck + 6, (ck + 6) % NIDX)
            wait_idx((ck + 2) % NIDX)
            issue_gathers((ck + 2) % NBUF, (ck + 2) % NIDX)
        wait_gather(db)
        compute(db)
        scatter(db, ib)
    for ck in range(NCHUNK - 4, NCHUNK):
        wait_scatter(ck % NBUF)

    plsc.subcore_barrier()
    pltpu.sync_copy(acc.at[pl.ds(s * RPT, RPT)],
                    out_hbm.at[c, pl.ds(s * RPT, RPT)])


def _tables(h, gal, gar, p1, p2, p3):
    el = jnp.dot(h, gal, preferred_element_type=jnp.float32)
    er = jnp.dot(h, gar, preferred_element_type=jnp.float32)
    ntab = (jnp.dot(h, p1, preferred_element_type=jnp.float32)
            + jnp.dot(el, p2, preferred_element_type=jnp.float32))
    ertab = jnp.dot(er, p3, preferred_element_type=jnp.float32)
    return ntab, ertab


def _tables0_body(x_ref, w_ref, gal_ref, gar_ref, p1_ref, p2_ref, p3_ref,
                  ntab_ref, ertab_ref):
    h = jnp.dot(x_ref[...], w_ref[...], preferred_element_type=jnp.float32)
    ntab, ertab = _tables(h, gal_ref[...], gar_ref[...],
                          p1_ref[...], p2_ref[...], p3_ref[...])
    ntab_ref[...] = ntab
    ertab_ref[...] = ertab


def _combine(p_ref, s_ref, t_ref, gt_ref):
    p = p_ref[0][0:N] + p_ref[1][0:N]
    hp = jnp.dot(p, s_ref[...], preferred_element_type=jnp.float32)
    den = jnp.dot(p, t_ref[...], preferred_element_type=jnp.float32)
    inv = 1.0 / (den + 1e-9)
    return hp * jnp.dot(inv, gt_ref[...], preferred_element_type=jnp.float32)


def _mid_body(p_ref, s_ref, t_ref, gt_ref, w_ref, gal_ref, gar_ref,
              p1_ref, p2_ref, p3_ref, ntab_ref, ertab_ref):
    x = _combine(p_ref, s_ref, t_ref, gt_ref)
    x = jnp.maximum(x, 0.01 * x)
    h = jnp.dot(x, w_ref[...], preferred_element_type=jnp.float32)
    ntab, ertab = _tables(h, gal_ref[...], gar_ref[...],
                          p1_ref[...], p2_ref[...], p3_ref[...])
    ntab_ref[...] = ntab
    ertab_ref[...] = ertab


def _final_body(p_ref, s_ref, t_ref, gt_ref, out_ref):
    out_ref[...] = _combine(p_ref, s_ref, t_ref, gt_ref)


_tables0_call = pl.pallas_call(
    _tables0_body,
    out_shape=[jax.ShapeDtypeStruct((N, ROW), jnp.float32),
               jax.ShapeDtypeStruct((N, ERW), jnp.float32)],
)

_mid_call = pl.pallas_call(
    _mid_body,
    out_shape=[jax.ShapeDtypeStruct((N, ROW), jnp.float32),
               jax.ShapeDtypeStruct((N, ERW), jnp.float32)],
)

_final_call = pl.pallas_call(
    _final_body,
    out_shape=jax.ShapeDtypeStruct((N, D), jnp.float32),
)


def _attn_mat(a):
    r = jnp.arange(D)
    return jnp.zeros((D, H), jnp.float32).at[r, r // DH].set(a.reshape(-1))


def kernel(n_feat, edge_index, e_feat, W0, a_l0, a_r0, W1, a_l1, a_r1):
    del e_feat
    src = edge_index[0].astype(jnp.int32).reshape(NW, NCHUNK, CH)
    dst = edge_index[1].astype(jnp.int32).reshape(NW, NCHUNK, CH)

    r = jnp.arange(D)
    r8 = jnp.arange(H)
    one = jnp.float32(1.0)
    p1 = jnp.zeros((D, ROW), jnp.float32).at[r, r].set(one)
    p2 = jnp.zeros((H, ROW), jnp.float32).at[r8, D + r8].set(one)
    p3 = jnp.zeros((H, ERW), jnp.float32).at[r8, r8].set(one)
    smat = jnp.zeros((ROW, D), jnp.float32).at[r, r].set(one)
    tmat = jnp.zeros((ROW, H), jnp.float32).at[D + r8, r8].set(one)
    gt = jnp.zeros((H, D), jnp.float32).at[r // DH, r].set(one)
    zeros_tab = jnp.zeros((NACC, ROW), jnp.float32)

    ntab, ertab = _tables0_call(n_feat, W0, _attn_mat(a_l0), _attn_mat(a_r0),
                                p1, p2, p3)
    part1 = _edge_kernel(src, dst, ntab, ertab, zeros_tab)
    ntab2, ertab2 = _mid_call(part1, smat, tmat, gt, W1,
                              _attn_mat(a_l1), _attn_mat(a_r1), p1, p2, p3)
    part2 = _edge_kernel(src, dst, ntab2, ertab2, zeros_tab)
    return _final_call(part2, smat, tmat, gt)


# speedup vs baseline: 201.6061x; 1.1301x over previous
import functools

import numpy as np

import jax
import jax.numpy as jnp
from jax import lax
from jax.experimental import pallas as pl
from jax.experimental.pallas import tpu as pltpu
from jax.experimental.pallas import tpu_sc as plsc

N = 10000
E = 320000
D = 128
H = 8
DH = 16
ERW = 16

NSC = 2
NTILE = 16
NW = NSC * NTILE
EPT = E // NW
CH = 40
NCHUNK = EPT // CH
NBUF = 4
NIDX = 8
RPT = N // NTILE


_MESH = plsc.VectorSubcoreMesh(core_axis_name="c", subcore_axis_name="s")


@functools.partial(
    pl.kernel,
    mesh=_MESH,
    compiler_params=pltpu.CompilerParams(use_tc_tiling_on_sc=False),
    out_type=[jax.ShapeDtypeStruct((NSC, N, D), jnp.float32),
              jax.ShapeDtypeStruct((NSC, N, ERW), jnp.float32)],
    scratch_types=[
        [pltpu.VMEM((CH,), jnp.int32) for _ in range(NIDX)],
        [pltpu.VMEM((CH,), jnp.int32) for _ in range(NIDX)],
        [pltpu.VMEM((CH, D), jnp.float32) for _ in range(NBUF)],
        [pltpu.VMEM((CH, ERW), jnp.float32) for _ in range(NBUF)],
        [pltpu.VMEM((CH, ERW), jnp.float32) for _ in range(NBUF)],
        [pltpu.VMEM((CH, ERW), jnp.float32) for _ in range(NBUF)],
        pltpu.VMEM_SHARED((N, D), jnp.float32),
        pltpu.VMEM_SHARED((N, ERW), jnp.float32),
        [pltpu.SemaphoreType.DMA for _ in range(NIDX)],
        [pltpu.SemaphoreType.DMA for _ in range(NBUF)],
        [pltpu.SemaphoreType.DMA for _ in range(NBUF)],
        [pltpu.SemaphoreType.DMA for _ in range(NBUF)],
        [pltpu.SemaphoreType.DMA for _ in range(NBUF)],
        [pltpu.SemaphoreType.DMA for _ in range(NBUF)],
    ],
)
def _edge_kernel(src_hbm, dst_hbm, htab_hbm, elr_hbm, z128_hbm, z16_hbm,
                 msg_hbm, ex_hbm,
                 srcbufs, dstbufs, rowbufs, elsbufs, eldbufs, exbufs,
                 acc, accex, isems, rsems, s1sems, s2sems, msems, xsems):
    c = lax.axis_index("c")
    s = lax.axis_index("s")
    wid = c * NTILE + s

    pltpu.sync_copy(z128_hbm.at[pl.ds(s * RPT, RPT)],
                    acc.at[pl.ds(s * RPT, RPT)])
    pltpu.sync_copy(z16_hbm.at[pl.ds(s * RPT, RPT)],
                    accex.at[pl.ds(s * RPT, RPT)])
    plsc.subcore_barrier()

    headmask = lax.iota(jnp.int32, 16) < H
    rot8 = lax.broadcast_in_dim(lax.iota(jnp.int32, 16) ^ 8, (16, 1), (0,))
    _dnums = lax.GatherDimensionNumbers(offset_dims=(),
                                        collapsed_slice_dims=(0,),
                                        start_index_map=(0,))

    def _gat16(vec, idx):
        return lax.gather(vec, idx, _dnums, slice_sizes=(1,),
                          mode=lax.GatherScatterMode.PROMISE_IN_BOUNDS)

    def issue_idx(k, ib):
        pltpu.async_copy(src_hbm.at[wid, k], srcbufs[ib], isems[ib])
        pltpu.async_copy(dst_hbm.at[wid, k], dstbufs[ib], isems[ib])

    def wait_idx(ib):
        pltpu.make_async_copy(src_hbm.at[wid, 0], srcbufs[ib],
                              isems[ib]).wait()
        pltpu.make_async_copy(dst_hbm.at[wid, 0], dstbufs[ib],
                              isems[ib]).wait()

    def issue_gathers(db, ib):
        pltpu.async_copy(htab_hbm.at[srcbufs[ib]], rowbufs[db], rsems[db])
        pltpu.async_copy(elr_hbm.at[srcbufs[ib]], elsbufs[db], s1sems[db])
        pltpu.async_copy(elr_hbm.at[dstbufs[ib]], eldbufs[db], s2sems[db])

    def wait_gather(db):
        pltpu.make_async_copy(htab_hbm.at[srcbufs[0]], rowbufs[db],
                              rsems[db]).wait()
        pltpu.make_async_copy(elr_hbm.at[srcbufs[0]], elsbufs[db],
                              s1sems[db]).wait()
        pltpu.make_async_copy(elr_hbm.at[dstbufs[0]], eldbufs[db],
                              s2sems[db]).wait()

    def scatter(db, ib):
        pltpu.async_copy(rowbufs[db], acc.at[dstbufs[ib]], msems[db],
                         add=True)
        pltpu.async_copy(exbufs[db], accex.at[dstbufs[ib]], xsems[db],
                         add=True)

    def wait_scatter(db):
        pltpu.make_async_copy(rowbufs[db], acc.at[dstbufs[0]],
                              msems[db]).wait()
        pltpu.make_async_copy(exbufs[db], accex.at[dstbufs[0]],
                              xsems[db]).wait()

    def compute(db):
        rowbuf = rowbufs[db]
        elsbuf = elsbufs[db]
        eldbuf = eldbufs[db]
        exbuf = exbufs[db]

        @plsc.parallel_loop(0, CH, unroll=4)
        def edge_body(i):
            a = elsbuf[i, :]
            bvec = eldbuf[i, :]
            rot = _gat16(bvec, rot8)
            ssum = a + rot
            e = jnp.maximum(ssum, 0.2 * ssum)
            ex = jnp.where(headmask, jnp.exp(e), 0.0)
            exbuf[i, :] = ex
            for hd in range(H):
                spl = _gat16(ex, jnp.full((16, 1), hd, jnp.int32))
                rowbuf[i, pl.ds(hd * DH, DH)] = rowbuf[i, pl.ds(hd * DH, DH)] * spl

    for k in range(6):
        issue_idx(k, k)
    for k in (0, 1):
        wait_idx(k)
        issue_gathers(k, k)
    for ck in (0, 1):
        issue_idx(ck + 6, ck + 6)
        wait_idx(ck + 2)
        issue_gathers(ck + 2, ck + 2)
        wait_gather(ck)
        compute(ck)
        scatter(ck, ck)

    n_groups = (NCHUNK - 2 - 8) // 8

    def group_body(g, carry):
        for j in range(8):
            ck = 2 + g * 8 + j
            db = (2 + j) % NBUF
            ib = (2 + j) % NIDX
            wait_scatter((2 + j + 2) % NBUF)
            issue_idx(ck + 6, (2 + j + 6) % NIDX)
            wait_idx((2 + j + 2) % NIDX)
            issue_gathers((2 + j + 2) % NBUF, (2 + j + 2) % NIDX)
            wait_gather(db)
            compute(db)
            scatter(db, ib)
        return carry

    lax.fori_loop(0, n_groups, group_body, 0)

    L = 2 + 8 * n_groups
    for ck in range(L, NCHUNK):
        db = ck % NBUF
        ib = ck % NIDX
        if ck + 2 < NCHUNK:
            wait_scatter((ck + 2) % NBUF)
            if ck + 6 < NCHUNK:
                issue_idx(ck + 6, (ck + 6) % NIDX)
            wait_idx((ck + 2) % NIDX)
            issue_gathers((ck + 2) % NBUF, (ck + 2) % NIDX)
        wait_gather(db)
        compute(db)
        scatter(db, ib)
    for ck in range(NCHUNK - 4, NCHUNK):
        wait_scatter(ck % NBUF)

    plsc.subcore_barrier()
    pltpu.sync_copy(acc.at[pl.ds(s * RPT, RPT)],
                    msg_hbm.at[c, pl.ds(s * RPT, RPT)])
    pltpu.sync_copy(accex.at[pl.ds(s * RPT, RPT)],
                    ex_hbm.at[c, pl.ds(s * RPT, RPT)])


def _tables(h, gal, gar, p3l, p3r):
    el = jnp.dot(h, gal, preferred_element_type=jnp.float32)
    er = jnp.dot(h, gar, preferred_element_type=jnp.float32)
    return (jnp.dot(el, p3l, preferred_element_type=jnp.float32)
            + jnp.dot(er, p3r, preferred_element_type=jnp.float32))


def _tables0_body(x_ref, w_ref, gal_ref, gar_ref, p3l_ref, p3r_ref,
                  h_ref, elr_ref):
    h = jnp.dot(x_ref[...], w_ref[...], preferred_element_type=jnp.float32)
    h_ref[...] = h
    elr_ref[...] = _tables(h, gal_ref[...], gar_ref[...],
                           p3l_ref[...], p3r_ref[...])


def _combine(pm_ref, px_ref, gt_ref):
    p = pm_ref[0] + pm_ref[1]
    den = px_ref[0] + px_ref[1]
    inv = 1.0 / (den + 1e-9)
    return p * jnp.dot(inv, gt_ref[...], preferred_element_type=jnp.float32)


def _mid_body(pm_ref, px_ref, gt_ref, w_ref, gal_ref, gar_ref,
              p3l_ref, p3r_ref, h_ref, elr_ref):
    x = _combine(pm_ref, px_ref, gt_ref)
    x = jnp.maximum(x, 0.01 * x)
    h = jnp.dot(x, w_ref[...], preferred_element_type=jnp.float32)
    h_ref[...] = h
    elr_ref[...] = _tables(h, gal_ref[...], gar_ref[...],
                           p3l_ref[...], p3r_ref[...])


def _final_body(pm_ref, px_ref, gt_ref, out_ref):
    out_ref[...] = _combine(pm_ref, px_ref, gt_ref)


_tables0_call = pl.pallas_call(
    _tables0_body,
    out_shape=[jax.ShapeDtypeStruct((N, D), jnp.float32),
               jax.ShapeDtypeStruct((N, ERW), jnp.float32)],
)

_mid_call = pl.pallas_call(
    _mid_body,
    out_shape=[jax.ShapeDtypeStruct((N, D), jnp.float32),
               jax.ShapeDtypeStruct((N, ERW), jnp.float32)],
)

_final_call = pl.pallas_call(
    _final_body,
    out_shape=jax.ShapeDtypeStruct((N, D), jnp.float32),
)


def _attn_mat(a):
    r = jnp.arange(D)
    return jnp.zeros((D, H), jnp.float32).at[r, r // DH].set(a.reshape(-1))


def kernel(n_feat, edge_index, e_feat, W0, a_l0, a_r0, W1, a_l1, a_r1):
    del e_feat
    src = edge_index[0].astype(jnp.int32).reshape(NW, NCHUNK, CH)
    dst = edge_index[1].astype(jnp.int32).reshape(NW, NCHUNK, CH)

    r = jnp.arange(D)
    r8 = jnp.arange(H)
    one = jnp.float32(1.0)
    p3l = jnp.zeros((H, ERW), jnp.float32).at[r8, r8].set(one)
    p3r = jnp.zeros((H, ERW), jnp.float32).at[r8, H + r8].set(one)
    gt16 = jnp.zeros((ERW, D), jnp.float32).at[r // DH, r].set(one)
    z128 = jnp.zeros((N, D), jnp.float32)
    z16 = jnp.zeros((N, ERW), jnp.float32)

    htab, elr = _tables0_call(n_feat, W0, _attn_mat(a_l0), _attn_mat(a_r0),
                              p3l, p3r)
    pm1, px1 = _edge_kernel(src, dst, htab, elr, z128, z16)
    htab2, elr2 = _mid_call(pm1, px1, gt16, W1,
                            _attn_mat(a_l1), _attn_mat(a_r1), p3l, p3r)
    pm2, px2 = _edge_kernel(src, dst, htab2, elr2, z128, z16)
    return _final_call(pm2, px2, gt16)
